# Initial kernel scaffold; baseline (speedup 1.0000x reference)
import jax
import jax.numpy as jnp
from jax.experimental import pallas as pl


def kernel(inputBatch, adj, W, a):
    raise NotImplementedError("write your pallas kernel here")


# speedup vs baseline: 2622.7424x; 2622.7424x over previous
import jax
import jax.numpy as jnp
from jax.experimental import pallas as pl
from jax.experimental.pallas import tpu as pltpu

_ALPHA = 0.2


def _gat_dense_kernel(x_ref, adj_ref, w_ref, a_ref, out_ref):
    x = x_ref[0]
    w = w_ref[...]
    h = jnp.dot(x, w, preferred_element_type=jnp.float32)
    h = jnp.where(jnp.isnan(h), 0.0, h)

    a = a_ref[...]
    out_f = w.shape[1]
    a1 = a[:, :out_f]
    a2 = a[:, out_f:]

    f = jax.lax.dot_general(h, a1, (((1,), (1,)), ((), ())),
                            preferred_element_type=jnp.float32)
    g = jax.lax.dot_general(a2, h, (((1,), (1,)), ((), ())),
                            preferred_element_type=jnp.float32)

    s = f + g
    s = jnp.where(s >= 0, s, _ALPHA * s)
    e = jnp.exp(-s)
    e = jnp.where(jnp.isnan(e), 0.0, e)
    e = jnp.where(adj_ref[...] != 0, e, 0.0)

    rowsum = jnp.sum(e, axis=1, keepdims=True)
    denom = jnp.where(rowsum != 0, rowsum, 1.0)

    hp = jnp.dot(e, h, preferred_element_type=jnp.float32) / denom
    hp = jnp.where(jnp.isnan(hp), 0.0, hp)
    hp = jnp.where(hp > 0, hp, jnp.exp(jnp.minimum(hp, 0.0)) - 1.0)
    out_ref[0] = hp


def kernel(inputBatch, adj, W, a):
    Bb, N, in_f = inputBatch.shape
    out_f = W.shape[1]
    return pl.pallas_call(
        _gat_dense_kernel,
        grid=(Bb,),
        in_specs=[
            pl.BlockSpec((1, N, in_f), lambda b: (b, 0, 0)),
            pl.BlockSpec((N, N), lambda b: (0, 0)),
            pl.BlockSpec((in_f, out_f), lambda b: (0, 0)),
            pl.BlockSpec((1, 2 * out_f), lambda b: (0, 0)),
        ],
        out_specs=pl.BlockSpec((1, N, out_f), lambda b: (b, 0, 0)),
        out_shape=jax.ShapeDtypeStruct((Bb, N, out_f), jnp.float32),
    )(inputBatch, adj, W, a)
